# baseline (speedup 1.0000x reference)
import functools

import jax
import jax.numpy as jnp
from jax import lax
from jax.experimental import pallas as pl
from jax.experimental.pallas import tpu as pltpu
from jax.experimental.pallas import tpu_sc as plsc

D = 768
S = 8192
B = 4
NROWS = B * S
NW = 32
ROWS_PER_W = NROWS // NW
SPANS_PER_BATCH = S // ROWS_PER_W
R = 32
NCHUNK = ROWS_PER_W // R
LG = D // 16


def _sc_call(x2d, st_rep, pe1d):
    mesh = plsc.VectorSubcoreMesh(core_axis_name="c", subcore_axis_name="s")

    @functools.partial(
        pl.kernel,
        mesh=mesh,
        out_type=jax.ShapeDtypeStruct((NROWS, D), jnp.float32),
        scratch_types=[
            pltpu.VMEM((R, D), jnp.float32),
            pltpu.VMEM((R, D), jnp.float32),
            pltpu.VMEM((R * D,), jnp.float32),
            pltpu.VMEM((R * D,), jnp.float32),
            pltpu.VMEM((D,), jnp.float32),
            pltpu.VMEM((16,), jnp.int32),
            pltpu.SemaphoreType.DMA,
            pltpu.SemaphoreType.DMA,
            pltpu.SemaphoreType.DMA,
            pltpu.SemaphoreType.DMA,
            pltpu.SemaphoreType.DMA,
            pltpu.SemaphoreType.DMA,
        ],
    )
    def k(x_hbm, st_hbm, pe_hbm, out_hbm, xb0, xb1, peb0, peb1, pe0,
          stv, semx0, semx1, semg0, semg1, semo0, semo1):
        cid = lax.axis_index("c")
        sid = lax.axis_index("s")
        wid = cid * 16 + sid
        batch = sid % B
        span = (sid // B) * 2 + cid
        s_base = span * ROWS_PER_W
        row_base = batch * S + s_base

        pltpu.sync_copy(st_hbm.at[wid], stv)
        pltpu.sync_copy(pe_hbm.at[pl.ds(0, D)], pe0)
        st = stv[...][0]

        xbufs = (xb0, xb1)
        pebufs = (peb0, peb1)
        semxs = (semx0, semx1)
        semgs = (semg0, semg1)
        semos = (semo0, semo1)

        def issue_in(g, slot):
            s0 = s_base + g * R
            pltpu.async_copy(x_hbm.at[pl.ds(row_base + g * R, R)],
                             xbufs[slot], semxs[slot])
            pre = s0 + R <= st
            src = jnp.maximum(s0 - st, 0) * D

            @pl.when(jnp.logical_not(pre))
            def _():
                pltpu.async_copy(pe_hbm.at[pl.ds(src, R * D)],
                                 pebufs[slot], semgs[slot])

        def process(g, slot):
            s0 = s_base + g * R
            pre = s0 + R <= st
            n_pre = jnp.clip(st - s0, 0, R)
            pltpu.make_async_copy(x_hbm.at[pl.ds(0, R)], xbufs[slot],
                                  semxs[slot]).wait()

            @pl.when(jnp.logical_not(pre))
            def _():
                pltpu.make_async_copy(pe_hbm.at[pl.ds(0, R * D)],
                                      pebufs[slot], semgs[slot]).wait()

                def row(i, c2):
                    q = jnp.maximum(i - n_pre, 0) * D
                    for j in range(LG):
                        plsc.addupdate(
                            xbufs[slot].at[i, pl.ds(j * 16, 16)],
                            pebufs[slot][pl.ds(q + j * 16, 16)])
                    return c2

                lax.fori_loop(0, R, row, 0)

            @pl.when(pre)
            def _():
                def row(i, c2):
                    for j in range(LG):
                        plsc.addupdate(
                            xbufs[slot].at[i, pl.ds(j * 16, 16)],
                            pe0[pl.ds(j * 16, 16)])
                    return c2

                lax.fori_loop(0, R, row, 0)

            pltpu.async_copy(xbufs[slot],
                             out_hbm.at[pl.ds(row_base + g * R, R)],
                             semos[slot])

        def drain_out(slot):
            pltpu.make_async_copy(xbufs[slot], out_hbm.at[pl.ds(0, R)],
                                  semos[slot]).wait()

        H = NCHUNK // 2
        issue_in(0, 0)

        def body(h, carry):
            g0 = 2 * h

            @pl.when(h > 0)
            def _():
                drain_out(1)

            issue_in(g0 + 1, 1)
            process(g0, 0)

            @pl.when(h < H - 1)
            def _():
                drain_out(0)
                issue_in(g0 + 2, 0)

            process(g0 + 1, 1)
            return carry

        lax.fori_loop(0, H, body, 0)
        drain_out(0)
        drain_out(1)

    return k(x2d, st_rep, pe1d)


def kernel(x, stidx, pe):
    x2d = x.reshape(NROWS, D)
    wbatch = (jnp.arange(NW) % 16) % B
    st_rep = jnp.broadcast_to(
        jnp.take(stidx.astype(jnp.int32), wbatch)[:, None], (NW, 16))
    st_rep = jnp.asarray(st_rep, jnp.int32) + jnp.zeros((NW, 16), jnp.int32)
    out = _sc_call(x2d, st_rep, pe.reshape(-1))
    return out.reshape(B, S, D)


# speedup vs baseline: 1.1211x; 1.1211x over previous
import functools

import jax
import jax.numpy as jnp
from jax import lax
from jax.experimental import pallas as pl
from jax.experimental.pallas import tpu as pltpu
from jax.experimental.pallas import tpu_sc as plsc

D = 768
S = 8192
B = 4
NROWS = B * S
NW = 32
ROWS_PER_W = NROWS // NW
SPANS_PER_BATCH = S // ROWS_PER_W
R = 16
NCHUNK = ROWS_PER_W // R
NSLOT = 4
LG = D // 16


def _sc_call(x2d, st_rep, pe1d):
    mesh = plsc.VectorSubcoreMesh(core_axis_name="c", subcore_axis_name="s")

    scratch = (
        [pltpu.VMEM((R, D), jnp.float32) for _ in range(NSLOT)]
        + [pltpu.VMEM((R * D,), jnp.float32) for _ in range(NSLOT)]
        + [pltpu.VMEM((D,), jnp.float32),
           pltpu.VMEM((16,), jnp.int32)]
        + [pltpu.SemaphoreType.DMA for _ in range(3 * NSLOT)]
    )

    @functools.partial(
        pl.kernel,
        mesh=mesh,
        out_type=jax.ShapeDtypeStruct((NROWS, D), jnp.float32),
        scratch_types=scratch,
    )
    def k(x_hbm, st_hbm, pe_hbm, out_hbm, *bufs):
        xbufs = bufs[0:NSLOT]
        pebufs = bufs[NSLOT:2 * NSLOT]
        pe0 = bufs[2 * NSLOT]
        stv = bufs[2 * NSLOT + 1]
        semxs = bufs[2 * NSLOT + 2:2 * NSLOT + 2 + NSLOT]
        semgs = bufs[2 * NSLOT + 2 + NSLOT:2 * NSLOT + 2 + 2 * NSLOT]
        semos = bufs[2 * NSLOT + 2 + 2 * NSLOT:2 * NSLOT + 2 + 3 * NSLOT]

        wid = lax.axis_index("c") * 16 + lax.axis_index("s")
        batch = wid // SPANS_PER_BATCH
        s_base = (wid % SPANS_PER_BATCH) * ROWS_PER_W
        row_base = batch * S + s_base

        pltpu.sync_copy(st_hbm.at[wid], stv)
        pltpu.sync_copy(pe_hbm.at[pl.ds(0, D)], pe0)
        st = stv[...][0]

        def issue_in(g, slot):
            s0 = s_base + g * R
            pltpu.async_copy(x_hbm.at[pl.ds(row_base + g * R, R)],
                             xbufs[slot], semxs[slot])
            pre = s0 + R <= st
            src = jnp.maximum(s0 - st, 0) * D

            @pl.when(jnp.logical_not(pre))
            def _():
                pltpu.async_copy(pe_hbm.at[pl.ds(src, R * D)],
                                 pebufs[slot], semgs[slot])

        def drain_out(slot):
            pltpu.make_async_copy(xbufs[slot], out_hbm.at[pl.ds(0, R)],
                                  semos[slot]).wait()

        def process(g, slot):
            s0 = s_base + g * R
            pre = s0 + R <= st
            n_pre = jnp.clip(st - s0, 0, R)
            pltpu.make_async_copy(x_hbm.at[pl.ds(0, R)], xbufs[slot],
                                  semxs[slot]).wait()

            @pl.when(jnp.logical_not(pre))
            def _():
                pltpu.make_async_copy(pe_hbm.at[pl.ds(0, R * D)],
                                      pebufs[slot], semgs[slot]).wait()

                def row(i, c2):
                    q = jnp.maximum(i - n_pre, 0) * D
                    for j in range(LG):
                        plsc.addupdate(
                            xbufs[slot].at[i, pl.ds(j * 16, 16)],
                            pebufs[slot][pl.ds(q + j * 16, 16)])
                    return c2

                lax.fori_loop(0, R, row, 0)

            @pl.when(pre)
            def _():
                def row(i, c2):
                    for j in range(LG):
                        plsc.addupdate(
                            xbufs[slot].at[i, pl.ds(j * 16, 16)],
                            pe0[pl.ds(j * 16, 16)])
                    return c2

                lax.fori_loop(0, R, row, 0)

            pltpu.async_copy(xbufs[slot],
                             out_hbm.at[pl.ds(row_base + g * R, R)],
                             semos[slot])

        issue_in(0, 0)
        issue_in(1, 1)

        def body(h, carry):
            for kk in range(NSLOT):
                g = NSLOT * h + kk
                slot = kk

                @pl.when(g >= 2)
                def _():
                    drain_out((slot + 2) % NSLOT)

                @pl.when(g + 2 < NCHUNK)
                def _():
                    issue_in(g + 2, (slot + 2) % NSLOT)

                process(g, slot)
            return carry

        lax.fori_loop(0, NCHUNK // NSLOT, body, 0)
        drain_out((NCHUNK - 2) % NSLOT)
        drain_out((NCHUNK - 1) % NSLOT)

    return k(x2d, st_rep, pe1d)


def kernel(x, stidx, pe):
    x2d = x.reshape(NROWS, D)
    st_rep = jnp.repeat(stidx.astype(jnp.int32),
                        SPANS_PER_BATCH * 16).reshape(NW, 16)
    out = _sc_call(x2d, st_rep, pe.reshape(-1))
    return out.reshape(B, S, D)
